# baseline (speedup 1.0000x reference)
import functools

import jax
import jax.numpy as jnp
from jax.experimental import pallas as pl
from jax.experimental.pallas import tpu as pltpu

D = 16
BN_EPS = 1e-5


def _msg_body(ea_ref, xj_ref, w1_ref, b1_ref, w2cat_ref, bmat_ref, out_ref):
    ea = ea_ref[...]
    xj = xj_ref[...]
    h = jnp.maximum(
        jnp.dot(ea, w1_ref[...], preferred_element_type=jnp.float32, precision=jax.lax.Precision.HIGHEST) + b1_ref[...],
        0.0)
    p = jnp.dot(xj, w2cat_ref[...], preferred_element_type=jnp.float32, precision=jax.lax.Precision.HIGHEST)
    acc = jnp.dot(xj, bmat_ref[...], preferred_element_type=jnp.float32, precision=jax.lax.Precision.HIGHEST)
    for k in range(D):
        acc = acc + h[:, k:k + 1] * p[:, k * D:(k + 1) * D]
    out_ref[...] = acc


def _msg_call(ea, xj, w1, b1, w2cat, bmat, tile):
    e = ea.shape[0]
    grid = e // tile
    full = lambda s: pl.BlockSpec(s, lambda i: (0, 0))
    return pl.pallas_call(
        _msg_body,
        grid=(grid,),
        in_specs=[
            pl.BlockSpec((tile, D), lambda i: (i, 0)),
            pl.BlockSpec((tile, D), lambda i: (i, 0)),
            full((D, D)),
            full((1, D)),
            full((D, D * D)),
            full((D, D)),
        ],
        out_specs=pl.BlockSpec((tile, D), lambda i: (i, 0)),
        out_shape=jax.ShapeDtypeStruct((e, D), jnp.float32),
    )(ea, xj, w1, b1, w2cat, bmat)


def _upd_body(n_nodes, extended, agg_ref, xp_ref, rootbd_ref, bias_ref,
              g_ref, bb_ref, s_ref, st_ref, *rest):
    t = jnp.sum(agg_ref[...], axis=0)
    t = t + jnp.dot(xp_ref[...], rootbd_ref[...],
                    preferred_element_type=jnp.float32, precision=jax.lax.Precision.HIGHEST) + bias_ref[...]
    t = jnp.maximum(t, 0.0)
    s = s_ref[...]
    st = st_ref[...]
    s1 = jnp.sum(t, axis=0, keepdims=True)
    m16 = jnp.dot(s1, s, preferred_element_type=jnp.float32, precision=jax.lax.Precision.HIGHEST) / n_nodes
    dev = t - jnp.dot(m16, st, preferred_element_type=jnp.float32, precision=jax.lax.Precision.HIGHEST)
    s2 = jnp.sum(dev * dev, axis=0, keepdims=True)
    var = jnp.dot(s2, s, preferred_element_type=jnp.float32, precision=jax.lax.Precision.HIGHEST) / n_nodes
    scale = g_ref[...] / jnp.sqrt(var + BN_EPS)
    shift = bb_ref[...] - scale * m16
    hb = (t * jnp.dot(scale, st, preferred_element_type=jnp.float32, precision=jax.lax.Precision.HIGHEST)
          + jnp.dot(shift, st, preferred_element_type=jnp.float32, precision=jax.lax.Precision.HIGHEST))
    if not extended:
        rest[-1][...] = hb
        return
    muw_ref, mub_ref, lvw_ref, lvb_ref, eps_ref, out_ref = rest
    mu = jnp.dot(hb, muw_ref[...], preferred_element_type=jnp.float32, precision=jax.lax.Precision.HIGHEST) + mub_ref[...]
    lv = jnp.minimum(
        jnp.dot(hb, lvw_ref[...], preferred_element_type=jnp.float32, precision=jax.lax.Precision.HIGHEST) + lvb_ref[...],
        10.0)
    out_ref[...] = mu + eps_ref[...] * jnp.exp(0.5 * lv)


def _upd_call(agg, xp, rootbd, bias, g, bb, s, st, extra=None):
    np_, w = xp.shape
    n_nodes = float(np_ * 8)
    args = [agg, xp, rootbd, bias, g, bb, s, st]
    if extra is not None:
        args += list(extra)
    body = functools.partial(_upd_body, n_nodes, extra is not None)
    return pl.pallas_call(
        body,
        out_shape=jax.ShapeDtypeStruct((np_, w), jnp.float32),
    )(*args)


def _dec_body(zs_ref, zd_ref, w0a_ref, w0b_ref, b0_ref, w1_ref, b1_ref,
              w2_ref, b2_ref, w3_ref, b3_ref, w4_ref, b4_ref, out_ref):
    d = jnp.maximum(
        jnp.dot(zs_ref[...], w0a_ref[...], preferred_element_type=jnp.float32, precision=jax.lax.Precision.HIGHEST)
        + jnp.dot(zd_ref[...], w0b_ref[...], preferred_element_type=jnp.float32, precision=jax.lax.Precision.HIGHEST)
        + b0_ref[...], 0.0)
    d = jnp.maximum(
        jnp.dot(d, w1_ref[...], preferred_element_type=jnp.float32, precision=jax.lax.Precision.HIGHEST) + b1_ref[...], 0.0)
    d = jnp.maximum(
        jnp.dot(d, w2_ref[...], preferred_element_type=jnp.float32, precision=jax.lax.Precision.HIGHEST) + b2_ref[...], 0.0)
    d = jnp.maximum(
        jnp.dot(d, w3_ref[...], preferred_element_type=jnp.float32, precision=jax.lax.Precision.HIGHEST) + b3_ref[...], 0.0)
    out_ref[...] = (
        jnp.dot(d, w4_ref[...], preferred_element_type=jnp.float32, precision=jax.lax.Precision.HIGHEST) + b4_ref[...])


def _dec_call(zs, zd, ws, tile):
    e = zs.shape[0]
    grid = e // tile
    specs = [pl.BlockSpec((tile, D), lambda i: (i, 0)),
             pl.BlockSpec((tile, D), lambda i: (i, 0))]
    for wgt in ws:
        specs.append(pl.BlockSpec(wgt.shape, lambda i: (0, 0)))
    return pl.pallas_call(
        _dec_body,
        grid=(grid,),
        in_specs=specs,
        out_specs=pl.BlockSpec((tile, D), lambda i: (i, 0)),
        out_shape=jax.ShapeDtypeStruct((e, D), jnp.float32),
    )(zs, zd, *ws)


def _blockdiag8(w):
    z = jnp.zeros((D, D), jnp.float32)
    rows = []
    for j in range(8):
        rows.append(jnp.concatenate([w if i == j else z for i in range(8)], axis=1))
    return jnp.concatenate(rows, axis=0)


def kernel(x, edge_index, edge_attr, params):
    n, _ = x.shape
    np_ = n // 8
    src = edge_index[0]
    dst = edge_index[1]
    p = params

    w2cat = p['nn_W2'].reshape(D, D, D).transpose(1, 0, 2).reshape(D, D * D)
    bmat = p['nn_b2'].reshape(D, D)
    b1 = p['nn_b1'].reshape(1, D)
    smat = jnp.tile(jnp.eye(D, dtype=jnp.float32), (8, 1))
    stmat = smat.T
    eps = jax.random.normal(jax.random.key(42), (n, D), jnp.float32)

    tile = 4000
    hp = x.reshape(np_, 8 * D)
    zp = None
    for i in (1, 2, 3, 4):
        h_nodes = hp.reshape(n, D)
        xj = jnp.take(h_nodes, src, axis=0)
        msg = _msg_call(edge_attr, xj, p['nn_W1'], b1, w2cat, bmat, tile)
        agg = jax.ops.segment_sum(msg, dst, num_segments=n)
        aggp = agg.reshape(1, np_, 8 * D)
        rootbd = _blockdiag8(p[f'root{i}'])
        bias = jnp.tile(p[f'bias{i}'], 8).reshape(1, 8 * D)
        g16 = p[f'bn{i}_g'].reshape(1, D)
        bb16 = p[f'bn{i}_b'].reshape(1, D)
        if i < 4:
            hp = _upd_call(aggp, hp, rootbd, bias, g16, bb16, smat, stmat)
        else:
            extra = (_blockdiag8(p['mu_W']), jnp.tile(p['mu_b'], 8).reshape(1, 8 * D),
                     _blockdiag8(p['lv_W']), jnp.tile(p['lv_b'], 8).reshape(1, 8 * D),
                     eps.reshape(np_, 8 * D))
            zp = _upd_call(aggp, hp, rootbd, bias, g16, bb16, smat, stmat, extra)

    z = zp.reshape(n, D)
    zs = jnp.take(z, src, axis=0)
    zd = jnp.take(z, dst, axis=0)
    ws = [p['dec_W0'][:D], p['dec_W0'][D:], p['dec_b0'].reshape(1, -1),
          p['dec_W1'], p['dec_b1'].reshape(1, -1),
          p['dec_W2'], p['dec_b2'].reshape(1, -1),
          p['dec_W3'], p['dec_b3'].reshape(1, -1),
          p['dec_W4'], p['dec_b4'].reshape(1, -1)]
    return _dec_call(zs, zd, ws, tile)


# speedup vs baseline: 1.1057x; 1.1057x over previous
import functools

import jax
import jax.numpy as jnp
from jax.experimental import pallas as pl
from jax.experimental.pallas import tpu as pltpu

D = 16
BN_EPS = 1e-5


def _msg_body(ea_ref, xj_ref, w1_ref, b1_ref, amat_ref, bmat2_ref, w2r_ref,
              bmat_ref, out_ref):
    hi = jax.lax.Precision.HIGHEST
    ea = ea_ref[...]
    xj = xj_ref[...]
    h = jnp.maximum(
        jnp.dot(ea, w1_ref[...], preferred_element_type=jnp.float32, precision=hi)
        + b1_ref[...], 0.0)
    g = (jnp.dot(h, amat_ref[...], preferred_element_type=jnp.float32, precision=hi)
         * jnp.dot(xj, bmat2_ref[...], preferred_element_type=jnp.float32, precision=hi))
    out_ref[...] = (
        jnp.dot(g, w2r_ref[...], preferred_element_type=jnp.float32, precision=hi)
        + jnp.dot(xj, bmat_ref[...], preferred_element_type=jnp.float32, precision=hi))


def _msg_call(ea, xj, w1, b1, amat, bmat2, w2r, bmat, tile):
    e = ea.shape[0]
    grid = e // tile
    full = lambda s: pl.BlockSpec(s, lambda i: (0, 0))
    return pl.pallas_call(
        _msg_body,
        grid=(grid,),
        in_specs=[
            pl.BlockSpec((tile, D), lambda i: (i, 0)),
            pl.BlockSpec((tile, D), lambda i: (i, 0)),
            full((D, D)),
            full((1, D)),
            full((D, D * D)),
            full((D, D * D)),
            full((D * D, D)),
            full((D, D)),
        ],
        out_specs=pl.BlockSpec((tile, D), lambda i: (i, 0)),
        out_shape=jax.ShapeDtypeStruct((e, D), jnp.float32),
    )(ea, xj, w1, b1, amat, bmat2, w2r, bmat)


def _upd_body(n_nodes, extended, agg_ref, xp_ref, rootbd_ref, bias_ref,
              g_ref, bb_ref, s_ref, st_ref, *rest):
    t = jnp.sum(agg_ref[...], axis=0)
    t = t + jnp.dot(xp_ref[...], rootbd_ref[...],
                    preferred_element_type=jnp.float32, precision=jax.lax.Precision.HIGHEST) + bias_ref[...]
    t = jnp.maximum(t, 0.0)
    s = s_ref[...]
    st = st_ref[...]
    s1 = jnp.sum(t, axis=0, keepdims=True)
    m16 = jnp.dot(s1, s, preferred_element_type=jnp.float32, precision=jax.lax.Precision.HIGHEST) / n_nodes
    dev = t - jnp.dot(m16, st, preferred_element_type=jnp.float32, precision=jax.lax.Precision.HIGHEST)
    s2 = jnp.sum(dev * dev, axis=0, keepdims=True)
    var = jnp.dot(s2, s, preferred_element_type=jnp.float32, precision=jax.lax.Precision.HIGHEST) / n_nodes
    scale = g_ref[...] / jnp.sqrt(var + BN_EPS)
    shift = bb_ref[...] - scale * m16
    hb = (t * jnp.dot(scale, st, preferred_element_type=jnp.float32, precision=jax.lax.Precision.HIGHEST)
          + jnp.dot(shift, st, preferred_element_type=jnp.float32, precision=jax.lax.Precision.HIGHEST))
    if not extended:
        rest[-1][...] = hb
        return
    muw_ref, mub_ref, lvw_ref, lvb_ref, eps_ref, out_ref = rest
    mu = jnp.dot(hb, muw_ref[...], preferred_element_type=jnp.float32, precision=jax.lax.Precision.HIGHEST) + mub_ref[...]
    lv = jnp.minimum(
        jnp.dot(hb, lvw_ref[...], preferred_element_type=jnp.float32, precision=jax.lax.Precision.HIGHEST) + lvb_ref[...],
        10.0)
    out_ref[...] = mu + eps_ref[...] * jnp.exp(0.5 * lv)


def _upd_call(agg, xp, rootbd, bias, g, bb, s, st, extra=None):
    np_, w = xp.shape
    n_nodes = float(np_ * 8)
    args = [agg, xp, rootbd, bias, g, bb, s, st]
    if extra is not None:
        args += list(extra)
    body = functools.partial(_upd_body, n_nodes, extra is not None)
    return pl.pallas_call(
        body,
        out_shape=jax.ShapeDtypeStruct((np_, w), jnp.float32),
    )(*args)


def _dec_body(zs_ref, zd_ref, w0a_ref, w0b_ref, b0_ref, w1_ref, b1_ref,
              w2_ref, b2_ref, w3_ref, b3_ref, w4_ref, b4_ref, out_ref):
    d = jnp.maximum(
        jnp.dot(zs_ref[...], w0a_ref[...], preferred_element_type=jnp.float32, precision=jax.lax.Precision.HIGHEST)
        + jnp.dot(zd_ref[...], w0b_ref[...], preferred_element_type=jnp.float32, precision=jax.lax.Precision.HIGHEST)
        + b0_ref[...], 0.0)
    d = jnp.maximum(
        jnp.dot(d, w1_ref[...], preferred_element_type=jnp.float32, precision=jax.lax.Precision.HIGHEST) + b1_ref[...], 0.0)
    d = jnp.maximum(
        jnp.dot(d, w2_ref[...], preferred_element_type=jnp.float32, precision=jax.lax.Precision.HIGHEST) + b2_ref[...], 0.0)
    d = jnp.maximum(
        jnp.dot(d, w3_ref[...], preferred_element_type=jnp.float32, precision=jax.lax.Precision.HIGHEST) + b3_ref[...], 0.0)
    out_ref[...] = (
        jnp.dot(d, w4_ref[...], preferred_element_type=jnp.float32, precision=jax.lax.Precision.HIGHEST) + b4_ref[...])


def _dec_call(zs, zd, ws, tile):
    e = zs.shape[0]
    grid = e // tile
    specs = [pl.BlockSpec((tile, D), lambda i: (i, 0)),
             pl.BlockSpec((tile, D), lambda i: (i, 0))]
    for wgt in ws:
        specs.append(pl.BlockSpec(wgt.shape, lambda i: (0, 0)))
    return pl.pallas_call(
        _dec_body,
        grid=(grid,),
        in_specs=specs,
        out_specs=pl.BlockSpec((tile, D), lambda i: (i, 0)),
        out_shape=jax.ShapeDtypeStruct((e, D), jnp.float32),
    )(zs, zd, *ws)


def _blockdiag8(w):
    z = jnp.zeros((D, D), jnp.float32)
    rows = []
    for j in range(8):
        rows.append(jnp.concatenate([w if i == j else z for i in range(8)], axis=1))
    return jnp.concatenate(rows, axis=0)


def kernel(x, edge_index, edge_attr, params):
    n, _ = x.shape
    np_ = n // 8
    src = edge_index[0]
    dst = edge_index[1]
    p = params

    w2r = p['nn_W2'].reshape(D * D, D)
    amat = jnp.repeat(jnp.eye(D, dtype=jnp.float32), D, axis=1)
    bmat2 = jnp.tile(jnp.eye(D, dtype=jnp.float32), (1, D))
    bmat = p['nn_b2'].reshape(D, D)
    b1 = p['nn_b1'].reshape(1, D)
    smat = jnp.tile(jnp.eye(D, dtype=jnp.float32), (8, 1))
    stmat = smat.T
    eps = jax.random.normal(jax.random.key(42), (n, D), jnp.float32)

    tile = 4000
    hp = x.reshape(np_, 8 * D)
    zp = None
    for i in (1, 2, 3, 4):
        h_nodes = hp.reshape(n, D)
        xj = jnp.take(h_nodes, src, axis=0)
        msg = _msg_call(edge_attr, xj, p['nn_W1'], b1, amat, bmat2, w2r, bmat, tile)
        agg = jax.ops.segment_sum(msg, dst, num_segments=n)
        aggp = agg.reshape(1, np_, 8 * D)
        rootbd = _blockdiag8(p[f'root{i}'])
        bias = jnp.tile(p[f'bias{i}'], 8).reshape(1, 8 * D)
        g16 = p[f'bn{i}_g'].reshape(1, D)
        bb16 = p[f'bn{i}_b'].reshape(1, D)
        if i < 4:
            hp = _upd_call(aggp, hp, rootbd, bias, g16, bb16, smat, stmat)
        else:
            extra = (_blockdiag8(p['mu_W']), jnp.tile(p['mu_b'], 8).reshape(1, 8 * D),
                     _blockdiag8(p['lv_W']), jnp.tile(p['lv_b'], 8).reshape(1, 8 * D),
                     eps.reshape(np_, 8 * D))
            zp = _upd_call(aggp, hp, rootbd, bias, g16, bb16, smat, stmat, extra)

    z = zp.reshape(n, D)
    zs = jnp.take(z, src, axis=0)
    zd = jnp.take(z, dst, axis=0)
    ws = [p['dec_W0'][:D], p['dec_W0'][D:], p['dec_b0'].reshape(1, -1),
          p['dec_W1'], p['dec_b1'].reshape(1, -1),
          p['dec_W2'], p['dec_b2'].reshape(1, -1),
          p['dec_W3'], p['dec_b3'].reshape(1, -1),
          p['dec_W4'], p['dec_b4'].reshape(1, -1)]
    return _dec_call(zs, zd, ws, tile)


# speedup vs baseline: 1.6647x; 1.5055x over previous
import functools

import jax
import jax.numpy as jnp
from jax.experimental import pallas as pl
from jax.experimental.pallas import tpu as pltpu

D = 16
BN_EPS = 1e-5
F32 = jnp.float32
BF16 = jnp.bfloat16
HI = jax.lax.Precision.HIGHEST


def _b16(x):
    return x.astype(BF16)


def _dotx(a, b):
    return jnp.dot(a, b, preferred_element_type=F32, precision=HI)


def _dot1(a16, b16):
    return jnp.dot(a16, b16, preferred_element_type=F32)


def _msg_body(ea_ref, xj_ref, w1_ref, b1_ref, w2b_ref, b2row_ref, arep_ref,
              csum_ref, out_ref):
    xj = xj_ref[...]
    h = jnp.maximum(_dot1(_b16(ea_ref[...]), w1_ref[...]) + b1_ref[...], 0.0)
    we = _b16(_dot1(_b16(h), w2b_ref[...]) + b2row_ref[...]).astype(F32)
    prod = _dot1(_b16(xj), arep_ref[...]) * we
    out_ref[...] = _dotx(prod, csum_ref[...])


def _msg_call(ea, xj, w1, b1, w2b, b2row, arep, csum, tile):
    e = ea.shape[0]
    grid = e // tile
    full = lambda a: pl.BlockSpec(a.shape, lambda i: tuple(0 for _ in a.shape))
    return pl.pallas_call(
        _msg_body,
        grid=(grid,),
        in_specs=[
            pl.BlockSpec((tile, D), lambda i: (i, 0)),
            pl.BlockSpec((tile, D), lambda i: (i, 0)),
            full(w1), full(b1), full(w2b), full(b2row), full(arep), full(csum),
        ],
        out_specs=pl.BlockSpec((tile, D), lambda i: (i, 0)),
        out_shape=jax.ShapeDtypeStruct((e, D), F32),
    )(ea, xj, w1, b1, w2b, b2row, arep, csum)


def _upd_body(n_nodes, extended, agg_ref, xp_ref, rootbd_ref, bias_ref,
              g_ref, bb_ref, s_ref, st_ref, *rest):
    t = jnp.sum(agg_ref[...], axis=0)
    t = t + _dot1(_b16(xp_ref[...]), rootbd_ref[...]) + bias_ref[...]
    t = jnp.maximum(t, 0.0)
    s = s_ref[...]
    st = st_ref[...]
    s1 = jnp.sum(t, axis=0, keepdims=True)
    m16 = _dotx(s1, s) / n_nodes
    dev = t - _dotx(m16, st)
    s2 = jnp.sum(dev * dev, axis=0, keepdims=True)
    var = _dotx(s2, s) / n_nodes
    scale = g_ref[...] / jnp.sqrt(var + BN_EPS)
    shift = bb_ref[...] - scale * m16
    hb = t * _dotx(scale, st) + _dotx(shift, st)
    if not extended:
        rest[-1][...] = hb
        return
    muw_ref, mub_ref, lvw_ref, lvb_ref, eps_ref, out_ref = rest
    hb16 = _b16(hb)
    mu = _dot1(hb16, muw_ref[...]) + mub_ref[...]
    lv = jnp.minimum(_dot1(hb16, lvw_ref[...]) + lvb_ref[...], 10.0)
    out_ref[...] = mu + eps_ref[...] * jnp.exp(0.5 * lv)


def _upd_call(agg, xp, rootbd, bias, g, bb, s, st, extra=None):
    np_, w = xp.shape
    n_nodes = float(np_ * 8)
    args = [agg, xp, rootbd, bias, g, bb, s, st]
    if extra is not None:
        args += list(extra)
    body = functools.partial(_upd_body, n_nodes, extra is not None)
    return pl.pallas_call(
        body,
        out_shape=jax.ShapeDtypeStruct((np_, w), F32),
    )(*args)


def _dec_body(zs_ref, zd_ref, w0a_ref, w0b_ref, b0_ref, w1_ref, b1_ref,
              w2_ref, b2_ref, w3_ref, b3_ref, w4_ref, b4_ref, out_ref):
    d = jnp.maximum(
        _dot1(_b16(zs_ref[...]), w0a_ref[...])
        + _dot1(_b16(zd_ref[...]), w0b_ref[...]) + b0_ref[...], 0.0)
    d = jnp.maximum(_dot1(_b16(d), w1_ref[...]) + b1_ref[...], 0.0)
    d = jnp.maximum(_dot1(_b16(d), w2_ref[...]) + b2_ref[...], 0.0)
    d = jnp.maximum(_dot1(_b16(d), w3_ref[...]) + b3_ref[...], 0.0)
    out_ref[...] = _dot1(_b16(d), w4_ref[...]) + b4_ref[...]


def _dec_call(zs, zd, ws, tile):
    e = zs.shape[0]
    grid = e // tile
    specs = [pl.BlockSpec((tile, D), lambda i: (i, 0)),
             pl.BlockSpec((tile, D), lambda i: (i, 0))]
    for wgt in ws:
        specs.append(pl.BlockSpec(wgt.shape, lambda i: tuple(0 for _ in wgt.shape)))
    return pl.pallas_call(
        _dec_body,
        grid=(grid,),
        in_specs=specs,
        out_specs=pl.BlockSpec((tile, D), lambda i: (i, 0)),
        out_shape=jax.ShapeDtypeStruct((e, D), F32),
    )(zs, zd, *ws)


def _blockdiag8(w):
    z = jnp.zeros((D, D), w.dtype)
    rows = []
    for j in range(8):
        rows.append(jnp.concatenate([w if i == j else z for i in range(8)], axis=1))
    return jnp.concatenate(rows, axis=0)


def kernel(x, edge_index, edge_attr, params):
    n, _ = x.shape
    np_ = n // 8
    src = edge_index[0]
    dst = edge_index[1]
    p = params

    w1b = _b16(p['nn_W1'])
    w2b = _b16(p['nn_W2'])
    b2row = p['nn_b2'].reshape(1, D * D)
    arep = jnp.repeat(jnp.eye(D, dtype=BF16), D, axis=1)
    csum = jnp.tile(jnp.eye(D, dtype=F32), (D, 1))
    b1 = p['nn_b1'].reshape(1, D)
    smat = jnp.tile(jnp.eye(D, dtype=F32), (8, 1))
    stmat = smat.T
    eps = jax.random.normal(jax.random.key(42), (n, D), F32)

    tile = 4000
    hp = x.reshape(np_, 8 * D)
    zp = None
    for i in (1, 2, 3, 4):
        h_nodes = hp.reshape(n, D)
        xj = jnp.take(h_nodes, src, axis=0)
        msg = _msg_call(edge_attr, xj, w1b, b1, w2b, b2row, arep, csum, tile)
        agg = jax.ops.segment_sum(msg, dst, num_segments=n)
        aggp = agg.reshape(1, np_, 8 * D)
        rootbd = _blockdiag8(_b16(p[f'root{i}']))
        bias = jnp.tile(p[f'bias{i}'], 8).reshape(1, 8 * D)
        g16 = p[f'bn{i}_g'].reshape(1, D)
        bb16 = p[f'bn{i}_b'].reshape(1, D)
        if i < 4:
            hp = _upd_call(aggp, hp, rootbd, bias, g16, bb16, smat, stmat)
        else:
            extra = (_blockdiag8(_b16(p['mu_W'])),
                     jnp.tile(p['mu_b'], 8).reshape(1, 8 * D),
                     _blockdiag8(_b16(p['lv_W'])),
                     jnp.tile(p['lv_b'], 8).reshape(1, 8 * D),
                     eps.reshape(np_, 8 * D))
            zp = _upd_call(aggp, hp, rootbd, bias, g16, bb16, smat, stmat, extra)

    z = zp.reshape(n, D)
    zs = jnp.take(z, src, axis=0)
    zd = jnp.take(z, dst, axis=0)
    ws = [_b16(p['dec_W0'][:D]), _b16(p['dec_W0'][D:]), p['dec_b0'].reshape(1, -1),
          _b16(p['dec_W1']), p['dec_b1'].reshape(1, -1),
          _b16(p['dec_W2']), p['dec_b2'].reshape(1, -1),
          _b16(p['dec_W3']), p['dec_b3'].reshape(1, -1),
          _b16(p['dec_W4']), p['dec_b4'].reshape(1, -1)]
    return _dec_call(zs, zd, ws, tile)


# speedup vs baseline: 3.9812x; 2.3916x over previous
import functools

import jax
import jax.numpy as jnp
from jax import lax
from jax.experimental import pallas as pl
from jax.experimental.pallas import tpu as pltpu
from jax.experimental.pallas import tpu_sc as plsc

NC = 2
NS = 16
CH = 128
GRP = 8

D = 16
BN_EPS = 1e-5
F32 = jnp.float32
BF16 = jnp.bfloat16
HI = jax.lax.Precision.HIGHEST


def _b16(x):
    return x.astype(BF16)


def _dotx(a, b):
    return jnp.dot(a, b, preferred_element_type=F32, precision=HI)


def _dot1(a16, b16):
    return jnp.dot(a16, b16, preferred_element_type=F32)


def _msg_body(ea_ref, xj_ref, w1_ref, b1_ref, w2b_ref, b2row_ref, arep_ref,
              csum_ref, out_ref):
    xj = xj_ref[...]
    h = jnp.maximum(_dot1(_b16(ea_ref[...]), w1_ref[...]) + b1_ref[...], 0.0)
    we = _b16(_dot1(_b16(h), w2b_ref[...]) + b2row_ref[...]).astype(F32)
    prod = _dot1(_b16(xj), arep_ref[...]) * we
    out_ref[...] = _dotx(prod, csum_ref[...])


def _msg_call(ea, xj, w1, b1, w2b, b2row, arep, csum, tile):
    e = ea.shape[0]
    grid = e // tile
    full = lambda a: pl.BlockSpec(a.shape, lambda i: tuple(0 for _ in a.shape))
    return pl.pallas_call(
        _msg_body,
        grid=(grid,),
        in_specs=[
            pl.BlockSpec((tile, D), lambda i: (i, 0)),
            pl.BlockSpec((tile, D), lambda i: (i, 0)),
            full(w1), full(b1), full(w2b), full(b2row), full(arep), full(csum),
        ],
        out_specs=pl.BlockSpec((tile, D), lambda i: (i, 0)),
        out_shape=jax.ShapeDtypeStruct((e, D), F32),
    )(ea, xj, w1, b1, w2b, b2row, arep, csum)


def _upd_body(n_nodes, extended, agg_ref, xp_ref, rootbd_ref, bias_ref,
              g_ref, bb_ref, s_ref, st_ref, *rest):
    t = jnp.sum(agg_ref[...], axis=0)
    t = t + _dot1(_b16(xp_ref[...]), rootbd_ref[...]) + bias_ref[...]
    t = jnp.maximum(t, 0.0)
    s = s_ref[...]
    st = st_ref[...]
    s1 = jnp.sum(t, axis=0, keepdims=True)
    m16 = _dotx(s1, s) / n_nodes
    dev = t - _dotx(m16, st)
    s2 = jnp.sum(dev * dev, axis=0, keepdims=True)
    var = _dotx(s2, s) / n_nodes
    scale = g_ref[...] / jnp.sqrt(var + BN_EPS)
    shift = bb_ref[...] - scale * m16
    hb = t * _dotx(scale, st) + _dotx(shift, st)
    if not extended:
        rest[-1][...] = hb
        return
    muw_ref, mub_ref, lvw_ref, lvb_ref, eps_ref, out_ref = rest
    hb16 = _b16(hb)
    mu = _dot1(hb16, muw_ref[...]) + mub_ref[...]
    lv = jnp.minimum(_dot1(hb16, lvw_ref[...]) + lvb_ref[...], 10.0)
    out_ref[...] = mu + eps_ref[...] * jnp.exp(0.5 * lv)


def _upd_call(agg, xp, rootbd, bias, g, bb, s, st, extra=None):
    np_, w = xp.shape
    n_nodes = float(np_ * 8)
    args = [agg, xp, rootbd, bias, g, bb, s, st]
    if extra is not None:
        args += list(extra)
    body = functools.partial(_upd_body, n_nodes, extra is not None)
    return pl.pallas_call(
        body,
        out_shape=jax.ShapeDtypeStruct((np_, w), F32),
    )(*args)


def _dec_body(zs_ref, zd_ref, w0a_ref, w0b_ref, b0_ref, w1_ref, b1_ref,
              w2_ref, b2_ref, w3_ref, b3_ref, w4_ref, b4_ref, out_ref):
    d = jnp.maximum(
        _dot1(_b16(zs_ref[...]), w0a_ref[...])
        + _dot1(_b16(zd_ref[...]), w0b_ref[...]) + b0_ref[...], 0.0)
    d = jnp.maximum(_dot1(_b16(d), w1_ref[...]) + b1_ref[...], 0.0)
    d = jnp.maximum(_dot1(_b16(d), w2_ref[...]) + b2_ref[...], 0.0)
    d = jnp.maximum(_dot1(_b16(d), w3_ref[...]) + b3_ref[...], 0.0)
    out_ref[...] = _dot1(_b16(d), w4_ref[...]) + b4_ref[...]


def _dec_call(zs, zd, ws, tile):
    e = zs.shape[0]
    grid = e // tile
    specs = [pl.BlockSpec((tile, D), lambda i: (i, 0)),
             pl.BlockSpec((tile, D), lambda i: (i, 0))]
    for wgt in ws:
        specs.append(pl.BlockSpec(wgt.shape, lambda i: tuple(0 for _ in wgt.shape)))
    return pl.pallas_call(
        _dec_body,
        grid=(grid,),
        in_specs=specs,
        out_specs=pl.BlockSpec((tile, D), lambda i: (i, 0)),
        out_shape=jax.ShapeDtypeStruct((e, D), F32),
    )(zs, zd, *ws)


def _sc_gather(table, idx2d):
    c_rows = idx2d.shape[0]
    k = c_rows // (NC * NS)
    grp_n = k // GRP
    mesh = plsc.VectorSubcoreMesh(core_axis_name="c", subcore_axis_name="s")

    @functools.partial(
        pl.kernel, mesh=mesh,
        compiler_params=pltpu.CompilerParams(use_tc_tiling_on_sc=False),
        out_type=jax.ShapeDtypeStruct((c_rows * CH, D), F32),
        scratch_types=[
            pltpu.VMEM((k, CH), jnp.int32),
            pltpu.VMEM((GRP * CH, D), F32),
            pltpu.SemaphoreType.DMA,
        ],
    )
    def kern(table_hbm, idx_hbm, out_hbm, idx_v, rows_v, sem):
        wid = lax.axis_index("s") * NC + lax.axis_index("c")
        base = wid * k
        pltpu.sync_copy(idx_hbm.at[pl.ds(base, k)], idx_v)

        def body(g, carry):
            cps = [pltpu.async_copy(
                table_hbm.at[idx_v.at[g * GRP + b]],
                rows_v.at[pl.ds(b * CH, CH)], sem) for b in range(GRP)]
            for cp in cps:
                cp.wait()
            pltpu.sync_copy(
                rows_v, out_hbm.at[pl.ds((base + g * GRP) * CH, GRP * CH)])
            return carry

        lax.fori_loop(0, grp_n, body, 0)

    return kern(table, idx2d)


def _sc_scatter_add(msg, idx2d, zrows, n_out):
    c_rows = idx2d.shape[0]
    k = c_rows // (NC * NS)
    grp_n = k // GRP
    nt = zrows.shape[0] * NS
    zk = zrows.shape[0]
    ok = n_out // NS
    mesh = plsc.VectorSubcoreMesh(core_axis_name="c", subcore_axis_name="s")

    @functools.partial(
        pl.kernel, mesh=mesh,
        compiler_params=pltpu.CompilerParams(use_tc_tiling_on_sc=False),
        out_type=jax.ShapeDtypeStruct((NC, n_out, D), F32),
        scratch_types=[
            pltpu.VMEM((k, CH), jnp.int32),
            pltpu.VMEM((GRP * CH, D), F32),
            pltpu.VMEM_SHARED((nt, D), F32),
            pltpu.SemaphoreType.DMA,
        ],
    )
    def kern(msg_hbm, idx_hbm, zro_hbm, out_hbm, idx_v, rows_v, shared, sem):
        c = lax.axis_index("c")
        s = lax.axis_index("s")
        base = c * (NS * k) + s * k
        pltpu.sync_copy(zro_hbm, shared.at[pl.ds(s * zk, zk)])
        pltpu.sync_copy(idx_hbm.at[pl.ds(base, k)], idx_v)
        plsc.subcore_barrier()

        def body(g, carry):
            pltpu.sync_copy(
                msg_hbm.at[pl.ds((base + g * GRP) * CH, GRP * CH)], rows_v)
            cps = [pltpu.async_copy(
                rows_v.at[pl.ds(b * CH, CH)],
                shared.at[idx_v.at[g * GRP + b]], sem, add=True)
                for b in range(GRP)]
            for cp in cps:
                cp.wait()
            return carry

        lax.fori_loop(0, grp_n, body, 0)
        plsc.subcore_barrier()
        pltpu.sync_copy(shared.at[pl.ds(s * ok, ok)],
                        out_hbm.at[c].at[pl.ds(s * ok, ok)])

    return kern(msg, idx2d, zrows)


def _blockdiag8(w):
    z = jnp.zeros((D, D), w.dtype)
    rows = []
    for j in range(8):
        rows.append(jnp.concatenate([w if i == j else z for i in range(8)], axis=1))
    return jnp.concatenate(rows, axis=0)


def kernel(x, edge_index, edge_attr, params):
    n, _ = x.shape
    np_ = n // 8
    e = edge_attr.shape[0]
    p = params

    step = NC * NS * GRP * CH
    epad = ((e + step - 1) // step) * step
    src = jnp.pad(edge_index[0], (0, epad - e))
    dst_s = jnp.pad(edge_index[1], (0, epad - e), constant_values=n)
    dst_g = jnp.pad(edge_index[1], (0, epad - e))
    idx_src = src.reshape(-1, CH)
    idx_dst_s = dst_s.reshape(-1, CH)
    idx_dst_g = dst_g.reshape(-1, CH)
    ea_pad = jnp.pad(edge_attr, ((0, epad - e), (0, 0)))
    nt_rows = ((n + NS * 8 - 1) // (NS * 8)) * 8
    zrows = jnp.zeros((nt_rows, D), F32)

    w1b = _b16(p['nn_W1'])
    w2b = _b16(p['nn_W2'])
    b2row = p['nn_b2'].reshape(1, D * D)
    arep = jnp.repeat(jnp.eye(D, dtype=BF16), D, axis=1)
    csum = jnp.tile(jnp.eye(D, dtype=F32), (D, 1))
    b1 = p['nn_b1'].reshape(1, D)
    smat = jnp.tile(jnp.eye(D, dtype=F32), (8, 1))
    stmat = smat.T
    eps = jax.random.normal(jax.random.key(42), (n, D), F32)

    tile = 4096
    hp = x.reshape(np_, 8 * D)
    zp = None
    for i in (1, 2, 3, 4):
        h_nodes = hp.reshape(n, D)
        xj = _sc_gather(h_nodes, idx_src)
        msg = _msg_call(ea_pad, xj, w1b, b1, w2b, b2row, arep, csum, tile)
        agg = _sc_scatter_add(msg, idx_dst_s, zrows, n)
        aggp = agg.reshape(NC, np_, 8 * D)
        rootbd = _blockdiag8(_b16(p[f'root{i}']))
        bias = jnp.tile(p[f'bias{i}'], 8).reshape(1, 8 * D)
        g16 = p[f'bn{i}_g'].reshape(1, D)
        bb16 = p[f'bn{i}_b'].reshape(1, D)
        if i < 4:
            hp = _upd_call(aggp, hp, rootbd, bias, g16, bb16, smat, stmat)
        else:
            extra = (_blockdiag8(_b16(p['mu_W'])),
                     jnp.tile(p['mu_b'], 8).reshape(1, 8 * D),
                     _blockdiag8(_b16(p['lv_W'])),
                     jnp.tile(p['lv_b'], 8).reshape(1, 8 * D),
                     eps.reshape(np_, 8 * D))
            zp = _upd_call(aggp, hp, rootbd, bias, g16, bb16, smat, stmat, extra)

    z = zp.reshape(n, D)
    zs = _sc_gather(z, idx_src)
    zd = _sc_gather(z, idx_dst_g)
    ws = [_b16(p['dec_W0'][:D]), _b16(p['dec_W0'][D:]), p['dec_b0'].reshape(1, -1),
          _b16(p['dec_W1']), p['dec_b1'].reshape(1, -1),
          _b16(p['dec_W2']), p['dec_b2'].reshape(1, -1),
          _b16(p['dec_W3']), p['dec_b3'].reshape(1, -1),
          _b16(p['dec_W4']), p['dec_b4'].reshape(1, -1)]
    return _dec_call(zs, zd, ws, tile)[:e]


# speedup vs baseline: 5.1829x; 1.3018x over previous
import functools

import jax
import jax.numpy as jnp
from jax import lax
from jax.experimental import pallas as pl
from jax.experimental.pallas import tpu as pltpu
from jax.experimental.pallas import tpu_sc as plsc

NC = 2
NS = 16
CH = 128
GRP = 8

D = 16
BN_EPS = 1e-5
F32 = jnp.float32
BF16 = jnp.bfloat16
HI = jax.lax.Precision.HIGHEST


def _b16(x):
    return x.astype(BF16)


def _dotx(a, b):
    return jnp.dot(a, b, preferred_element_type=F32, precision=HI)


def _dot1(a16, b16):
    return jnp.dot(a16, b16, preferred_element_type=F32)


def _dot_split(a, b16mat):
    hi = _b16(a)
    lo = _b16(a - hi.astype(F32))
    return _dot1(hi, b16mat) + _dot1(lo, b16mat)


def _we_body(ea_ref, w1_ref, b1_ref, w2b_ref, b2row_ref, out_ref):
    h = jnp.maximum(_dot1(_b16(ea_ref[...]), w1_ref[...]) + b1_ref[...], 0.0)
    out_ref[...] = _b16(_dot1(_b16(h), w2b_ref[...]) + b2row_ref[...])


def _we_call(ea, w1, b1, w2b, b2row, tile):
    e = ea.shape[0]
    grid = e // tile
    full = lambda a: pl.BlockSpec(a.shape, lambda i: tuple(0 for _ in a.shape))
    return pl.pallas_call(
        _we_body,
        grid=(grid,),
        in_specs=[
            pl.BlockSpec((tile, D), lambda i: (i, 0)),
            full(w1), full(b1), full(w2b), full(b2row),
        ],
        out_specs=pl.BlockSpec((tile, D * D), lambda i: (i, 0)),
        out_shape=jax.ShapeDtypeStruct((e, D * D), BF16),
    )(ea, w1, b1, w2b, b2row)


def _msg_body(we_ref, xj_ref, arep_ref, csum_ref, out_ref):
    prod = _dot1(_b16(xj_ref[...]), arep_ref[...]) * we_ref[...].astype(F32)
    out_ref[...] = _dot_split(prod, csum_ref[...])


def _msg_call(we, xj, arep, csum, tile):
    e = we.shape[0]
    grid = e // tile
    full = lambda a: pl.BlockSpec(a.shape, lambda i: tuple(0 for _ in a.shape))
    return pl.pallas_call(
        _msg_body,
        grid=(grid,),
        in_specs=[
            pl.BlockSpec((tile, D * D), lambda i: (i, 0)),
            pl.BlockSpec((tile, D), lambda i: (i, 0)),
            full(arep), full(csum),
        ],
        out_specs=pl.BlockSpec((tile, D), lambda i: (i, 0)),
        out_shape=jax.ShapeDtypeStruct((e, D), F32),
    )(we, xj, arep, csum)


def _upd_body(n_nodes, extended, agg_ref, xp_ref, rootbd_ref, bias_ref,
              g_ref, bb_ref, s_ref, st_ref, *rest):
    t = jnp.sum(agg_ref[...], axis=0)
    t = t + _dot1(_b16(xp_ref[...]), rootbd_ref[...]) + bias_ref[...]
    t = jnp.maximum(t, 0.0)
    s = s_ref[...]
    st = st_ref[...]
    s1 = jnp.sum(t, axis=0, keepdims=True)
    m16 = _dotx(s1, s) / n_nodes
    dev = t - _dotx(m16, st)
    s2 = jnp.sum(dev * dev, axis=0, keepdims=True)
    var = _dotx(s2, s) / n_nodes
    scale = g_ref[...] / jnp.sqrt(var + BN_EPS)
    shift = bb_ref[...] - scale * m16
    hb = t * _dotx(scale, st) + _dotx(shift, st)
    if not extended:
        rest[-1][...] = hb
        return
    muw_ref, mub_ref, lvw_ref, lvb_ref, eps_ref, out_ref = rest
    hb16 = _b16(hb)
    mu = _dot1(hb16, muw_ref[...]) + mub_ref[...]
    lv = jnp.minimum(_dot1(hb16, lvw_ref[...]) + lvb_ref[...], 10.0)
    out_ref[...] = mu + eps_ref[...] * jnp.exp(0.5 * lv)


def _upd_call(agg, xp, rootbd, bias, g, bb, s, st, extra=None):
    np_, w = xp.shape
    n_nodes = float(np_ * 8)
    args = [agg, xp, rootbd, bias, g, bb, s, st]
    if extra is not None:
        args += list(extra)
    body = functools.partial(_upd_body, n_nodes, extra is not None)
    return pl.pallas_call(
        body,
        out_shape=jax.ShapeDtypeStruct((np_, w), F32),
    )(*args)


def _dec_body(zs_ref, zd_ref, w0a_ref, w0b_ref, b0_ref, w1_ref, b1_ref,
              w2_ref, b2_ref, w3_ref, b3_ref, w4_ref, b4_ref, out_ref):
    d = jnp.maximum(
        _dot1(_b16(zs_ref[...]), w0a_ref[...])
        + _dot1(_b16(zd_ref[...]), w0b_ref[...]) + b0_ref[...], 0.0)
    d = jnp.maximum(_dot1(_b16(d), w1_ref[...]) + b1_ref[...], 0.0)
    d = jnp.maximum(_dot1(_b16(d), w2_ref[...]) + b2_ref[...], 0.0)
    d = jnp.maximum(_dot1(_b16(d), w3_ref[...]) + b3_ref[...], 0.0)
    out_ref[...] = _dot1(_b16(d), w4_ref[...]) + b4_ref[...]


def _dec_call(zs, zd, ws, tile):
    e = zs.shape[0]
    grid = e // tile
    specs = [pl.BlockSpec((tile, D), lambda i: (i, 0)),
             pl.BlockSpec((tile, D), lambda i: (i, 0))]
    for wgt in ws:
        specs.append(pl.BlockSpec(wgt.shape, lambda i: tuple(0 for _ in wgt.shape)))
    return pl.pallas_call(
        _dec_body,
        grid=(grid,),
        in_specs=specs,
        out_specs=pl.BlockSpec((tile, D), lambda i: (i, 0)),
        out_shape=jax.ShapeDtypeStruct((e, D), F32),
    )(zs, zd, *ws)


def _sc_gather(table, idx2d):
    c_rows = idx2d.shape[0]
    k = c_rows // (NC * NS)
    grp_n = k // GRP
    mesh = plsc.VectorSubcoreMesh(core_axis_name="c", subcore_axis_name="s")

    @functools.partial(
        pl.kernel, mesh=mesh,
        compiler_params=pltpu.CompilerParams(use_tc_tiling_on_sc=False),
        out_type=jax.ShapeDtypeStruct((c_rows * CH, D), F32),
        scratch_types=[
            pltpu.VMEM((k, CH), jnp.int32),
            pltpu.VMEM((GRP * CH, D), F32),
            pltpu.SemaphoreType.DMA,
        ],
    )
    def kern(table_hbm, idx_hbm, out_hbm, idx_v, rows_v, sem):
        wid = lax.axis_index("s") * NC + lax.axis_index("c")
        base = wid * k
        pltpu.sync_copy(idx_hbm.at[pl.ds(base, k)], idx_v)

        def body(g, carry):
            cps = [pltpu.async_copy(
                table_hbm.at[idx_v.at[g * GRP + b]],
                rows_v.at[pl.ds(b * CH, CH)], sem) for b in range(GRP)]
            for cp in cps:
                cp.wait()
            pltpu.sync_copy(
                rows_v, out_hbm.at[pl.ds((base + g * GRP) * CH, GRP * CH)])
            return carry

        lax.fori_loop(0, grp_n, body, 0)

    return kern(table, idx2d)


def _sc_scatter_add(msg, idx2d, zrows, n_out):
    c_rows = idx2d.shape[0]
    k = c_rows // (NC * NS)
    grp_n = k // GRP
    nt = zrows.shape[0] * NS
    zk = zrows.shape[0]
    ok = n_out // NS
    mesh = plsc.VectorSubcoreMesh(core_axis_name="c", subcore_axis_name="s")

    @functools.partial(
        pl.kernel, mesh=mesh,
        compiler_params=pltpu.CompilerParams(use_tc_tiling_on_sc=False),
        out_type=jax.ShapeDtypeStruct((NC, n_out, D), F32),
        scratch_types=[
            pltpu.VMEM((k, CH), jnp.int32),
            pltpu.VMEM((GRP * CH, D), F32),
            pltpu.VMEM_SHARED((nt, D), F32),
            pltpu.SemaphoreType.DMA,
        ],
    )
    def kern(msg_hbm, idx_hbm, zro_hbm, out_hbm, idx_v, rows_v, shared, sem):
        c = lax.axis_index("c")
        s = lax.axis_index("s")
        base = c * (NS * k) + s * k
        pltpu.sync_copy(zro_hbm, shared.at[pl.ds(s * zk, zk)])
        pltpu.sync_copy(idx_hbm.at[pl.ds(base, k)], idx_v)
        plsc.subcore_barrier()

        def body(g, carry):
            pltpu.sync_copy(
                msg_hbm.at[pl.ds((base + g * GRP) * CH, GRP * CH)], rows_v)
            cps = [pltpu.async_copy(
                rows_v.at[pl.ds(b * CH, CH)],
                shared.at[idx_v.at[g * GRP + b]], sem, add=True)
                for b in range(GRP)]
            for cp in cps:
                cp.wait()
            return carry

        lax.fori_loop(0, grp_n, body, 0)
        plsc.subcore_barrier()
        pltpu.sync_copy(shared.at[pl.ds(s * ok, ok)],
                        out_hbm.at[c].at[pl.ds(s * ok, ok)])

    return kern(msg, idx2d, zrows)


def _blockdiag8(w):
    z = jnp.zeros((D, D), w.dtype)
    rows = []
    for j in range(8):
        rows.append(jnp.concatenate([w if i == j else z for i in range(8)], axis=1))
    return jnp.concatenate(rows, axis=0)


def kernel(x, edge_index, edge_attr, params):
    n, _ = x.shape
    np_ = n // 8
    e = edge_attr.shape[0]
    p = params

    step = NC * NS * GRP * CH
    epad = ((e + step - 1) // step) * step
    src = jnp.pad(edge_index[0], (0, epad - e))
    dst_s = jnp.pad(edge_index[1], (0, epad - e), constant_values=n)
    dst_g = jnp.pad(edge_index[1], (0, epad - e))
    idx_src = src.reshape(-1, CH)
    idx_dst_s = dst_s.reshape(-1, CH)
    idx_dst_g = dst_g.reshape(-1, CH)
    ea_pad = jnp.pad(edge_attr, ((0, epad - e), (0, 0)))
    nt_rows = ((n + NS * 8 - 1) // (NS * 8)) * 8
    zrows = jnp.zeros((nt_rows, D), F32)

    w1b = _b16(p['nn_W1'])
    w2b = _b16(p['nn_W2'])
    b2row = p['nn_b2'].reshape(1, D * D)
    arep = jnp.repeat(jnp.eye(D, dtype=BF16), D, axis=1)
    csum = jnp.tile(jnp.eye(D, dtype=BF16), (D, 1))
    b1 = p['nn_b1'].reshape(1, D)
    smat = jnp.tile(jnp.eye(D, dtype=F32), (8, 1))
    stmat = smat.T
    eps = jax.random.normal(jax.random.key(42), (n, D), F32)

    tile = 4096
    hp = x.reshape(np_, 8 * D)
    zp = None
    we = _we_call(ea_pad, w1b, b1, w2b, b2row, tile)
    for i in (1, 2, 3, 4):
        h_nodes = hp.reshape(n, D)
        xj = _sc_gather(h_nodes, idx_src)
        msg = _msg_call(we, xj, arep, csum, tile)
        agg = _sc_scatter_add(msg, idx_dst_s, zrows, n)
        aggp = agg.reshape(NC, np_, 8 * D)
        rootbd = _blockdiag8(_b16(p[f'root{i}']))
        bias = jnp.tile(p[f'bias{i}'], 8).reshape(1, 8 * D)
        g16 = p[f'bn{i}_g'].reshape(1, D)
        bb16 = p[f'bn{i}_b'].reshape(1, D)
        if i < 4:
            hp = _upd_call(aggp, hp, rootbd, bias, g16, bb16, smat, stmat)
        else:
            extra = (_blockdiag8(_b16(p['mu_W'])),
                     jnp.tile(p['mu_b'], 8).reshape(1, 8 * D),
                     _blockdiag8(_b16(p['lv_W'])),
                     jnp.tile(p['lv_b'], 8).reshape(1, 8 * D),
                     eps.reshape(np_, 8 * D))
            zp = _upd_call(aggp, hp, rootbd, bias, g16, bb16, smat, stmat, extra)

    z = zp.reshape(n, D)
    zs = _sc_gather(z, idx_src)
    zd = _sc_gather(z, idx_dst_g)
    ws = [_b16(p['dec_W0'][:D]), _b16(p['dec_W0'][D:]), p['dec_b0'].reshape(1, -1),
          _b16(p['dec_W1']), p['dec_b1'].reshape(1, -1),
          _b16(p['dec_W2']), p['dec_b2'].reshape(1, -1),
          _b16(p['dec_W3']), p['dec_b3'].reshape(1, -1),
          _b16(p['dec_W4']), p['dec_b4'].reshape(1, -1)]
    return _dec_call(zs, zd, ws, tile)[:e]


# speedup vs baseline: 5.1965x; 1.0026x over previous
import functools

import jax
import jax.numpy as jnp
from jax import lax
from jax.experimental import pallas as pl
from jax.experimental.pallas import tpu as pltpu
from jax.experimental.pallas import tpu_sc as plsc

NC = 2
NS = 16
CH = 128
GRP = 20

D = 16
BN_EPS = 1e-5
F32 = jnp.float32
BF16 = jnp.bfloat16
HI = jax.lax.Precision.HIGHEST


def _b16(x):
    return x.astype(BF16)


def _dotx(a, b):
    return jnp.dot(a, b, preferred_element_type=F32, precision=HI)


def _dot1(a16, b16):
    return jnp.dot(a16, b16, preferred_element_type=F32)


def _dot_split(a, b16mat):
    hi = _b16(a)
    lo = _b16(a - hi.astype(F32))
    return _dot1(hi, b16mat) + _dot1(lo, b16mat)


def _we_body(ea_ref, w1_ref, b1_ref, w2b_ref, b2row_ref, out_ref):
    h = jnp.maximum(_dot1(_b16(ea_ref[...]), w1_ref[...]) + b1_ref[...], 0.0)
    out_ref[...] = _b16(_dot1(_b16(h), w2b_ref[...]) + b2row_ref[...])


def _we_call(ea, w1, b1, w2b, b2row, tile):
    e = ea.shape[0]
    grid = e // tile
    full = lambda a: pl.BlockSpec(a.shape, lambda i: tuple(0 for _ in a.shape))
    return pl.pallas_call(
        _we_body,
        grid=(grid,),
        in_specs=[
            pl.BlockSpec((tile, D), lambda i: (i, 0)),
            full(w1), full(b1), full(w2b), full(b2row),
        ],
        out_specs=pl.BlockSpec((tile, D * D), lambda i: (i, 0)),
        out_shape=jax.ShapeDtypeStruct((e, D * D), BF16),
    )(ea, w1, b1, w2b, b2row)


def _msg_body(we_ref, xj_ref, arep_ref, csum_ref, out_ref):
    prod = _dot1(_b16(xj_ref[...]), arep_ref[...]) * we_ref[...].astype(F32)
    out_ref[...] = _dot_split(prod, csum_ref[...])


def _msg_call(we, xj, arep, csum, tile):
    e = we.shape[0]
    grid = e // tile
    full = lambda a: pl.BlockSpec(a.shape, lambda i: tuple(0 for _ in a.shape))
    return pl.pallas_call(
        _msg_body,
        grid=(grid,),
        in_specs=[
            pl.BlockSpec((tile, D * D), lambda i: (i, 0)),
            pl.BlockSpec((tile, D), lambda i: (i, 0)),
            full(arep), full(csum),
        ],
        out_specs=pl.BlockSpec((tile, D), lambda i: (i, 0)),
        out_shape=jax.ShapeDtypeStruct((e, D), F32),
    )(we, xj, arep, csum)


def _upd_body(n_nodes, extended, agg_ref, xp_ref, rootbd_ref, bias_ref,
              g_ref, bb_ref, s_ref, st_ref, *rest):
    t = jnp.sum(agg_ref[...], axis=0)
    t = t + _dot1(_b16(xp_ref[...]), rootbd_ref[...]) + bias_ref[...]
    t = jnp.maximum(t, 0.0)
    s = s_ref[...]
    st = st_ref[...]
    s1 = jnp.sum(t, axis=0, keepdims=True)
    m16 = _dotx(s1, s) / n_nodes
    dev = t - _dotx(m16, st)
    s2 = jnp.sum(dev * dev, axis=0, keepdims=True)
    var = _dotx(s2, s) / n_nodes
    scale = g_ref[...] / jnp.sqrt(var + BN_EPS)
    shift = bb_ref[...] - scale * m16
    hb = t * _dotx(scale, st) + _dotx(shift, st)
    if not extended:
        rest[-1][...] = hb
        return
    muw_ref, mub_ref, lvw_ref, lvb_ref, eps_ref, out_ref = rest
    hb16 = _b16(hb)
    mu = _dot1(hb16, muw_ref[...]) + mub_ref[...]
    lv = jnp.minimum(_dot1(hb16, lvw_ref[...]) + lvb_ref[...], 10.0)
    out_ref[...] = mu + eps_ref[...] * jnp.exp(0.5 * lv)


def _upd_call(agg, xp, rootbd, bias, g, bb, s, st, extra=None):
    np_, w = xp.shape
    n_nodes = float(np_ * 8)
    args = [agg, xp, rootbd, bias, g, bb, s, st]
    if extra is not None:
        args += list(extra)
    body = functools.partial(_upd_body, n_nodes, extra is not None)
    return pl.pallas_call(
        body,
        out_shape=jax.ShapeDtypeStruct((np_, w), F32),
    )(*args)


def _dec_body(zs_ref, zd_ref, w0a_ref, w0b_ref, b0_ref, w1_ref, b1_ref,
              w2_ref, b2_ref, w3_ref, b3_ref, w4_ref, b4_ref, out_ref):
    d = jnp.maximum(
        _dot1(_b16(zs_ref[...]), w0a_ref[...])
        + _dot1(_b16(zd_ref[...]), w0b_ref[...]) + b0_ref[...], 0.0)
    d = jnp.maximum(_dot1(_b16(d), w1_ref[...]) + b1_ref[...], 0.0)
    d = jnp.maximum(_dot1(_b16(d), w2_ref[...]) + b2_ref[...], 0.0)
    d = jnp.maximum(_dot1(_b16(d), w3_ref[...]) + b3_ref[...], 0.0)
    out_ref[...] = _dot1(_b16(d), w4_ref[...]) + b4_ref[...]


def _dec_call(zs, zd, ws, tile):
    e = zs.shape[0]
    grid = e // tile
    specs = [pl.BlockSpec((tile, D), lambda i: (i, 0)),
             pl.BlockSpec((tile, D), lambda i: (i, 0))]
    for wgt in ws:
        specs.append(pl.BlockSpec(wgt.shape, lambda i: tuple(0 for _ in wgt.shape)))
    return pl.pallas_call(
        _dec_body,
        grid=(grid,),
        in_specs=specs,
        out_specs=pl.BlockSpec((tile, D), lambda i: (i, 0)),
        out_shape=jax.ShapeDtypeStruct((e, D), F32),
    )(zs, zd, *ws)


def _sc_gather(table, idx2d):
    c_rows = idx2d.shape[0]
    k = c_rows // (NC * NS)
    grp_n = k // GRP
    mesh = plsc.VectorSubcoreMesh(core_axis_name="c", subcore_axis_name="s")

    @functools.partial(
        pl.kernel, mesh=mesh,
        compiler_params=pltpu.CompilerParams(use_tc_tiling_on_sc=False),
        out_type=jax.ShapeDtypeStruct((c_rows * CH, D), F32),
        scratch_types=[
            pltpu.VMEM((k, CH), jnp.int32),
            pltpu.VMEM((GRP * CH, D), F32),
            pltpu.SemaphoreType.DMA,
        ],
    )
    def kern(table_hbm, idx_hbm, out_hbm, idx_v, rows_v, sem):
        wid = lax.axis_index("s") * NC + lax.axis_index("c")
        base = wid * k
        pltpu.sync_copy(idx_hbm.at[pl.ds(base, k)], idx_v)

        def body(g, carry):
            cps = [pltpu.async_copy(
                table_hbm.at[idx_v.at[g * GRP + b]],
                rows_v.at[pl.ds(b * CH, CH)], sem) for b in range(GRP)]
            for cp in cps:
                cp.wait()
            pltpu.sync_copy(
                rows_v, out_hbm.at[pl.ds((base + g * GRP) * CH, GRP * CH)])
            return carry

        lax.fori_loop(0, grp_n, body, 0)

    return kern(table, idx2d)


def _sc_scatter_add(msg, idx2d, zrows, n_out):
    c_rows = idx2d.shape[0]
    k = c_rows // (NC * NS)
    grp_n = k // GRP
    nt = zrows.shape[0] * NS
    zk = zrows.shape[0]
    ok = n_out // NS
    mesh = plsc.VectorSubcoreMesh(core_axis_name="c", subcore_axis_name="s")

    @functools.partial(
        pl.kernel, mesh=mesh,
        compiler_params=pltpu.CompilerParams(use_tc_tiling_on_sc=False),
        out_type=jax.ShapeDtypeStruct((NC, n_out, D), F32),
        scratch_types=[
            pltpu.VMEM((k, CH), jnp.int32),
            pltpu.VMEM((GRP * CH, D), F32),
            pltpu.VMEM_SHARED((nt, D), F32),
            pltpu.SemaphoreType.DMA,
        ],
    )
    def kern(msg_hbm, idx_hbm, zro_hbm, out_hbm, idx_v, rows_v, shared, sem):
        c = lax.axis_index("c")
        s = lax.axis_index("s")
        base = c * (NS * k) + s * k
        pltpu.sync_copy(zro_hbm, shared.at[pl.ds(s * zk, zk)])
        pltpu.sync_copy(idx_hbm.at[pl.ds(base, k)], idx_v)
        plsc.subcore_barrier()

        def body(g, carry):
            pltpu.sync_copy(
                msg_hbm.at[pl.ds((base + g * GRP) * CH, GRP * CH)], rows_v)
            cps = [pltpu.async_copy(
                rows_v.at[pl.ds(b * CH, CH)],
                shared.at[idx_v.at[g * GRP + b]], sem, add=True)
                for b in range(GRP)]
            for cp in cps:
                cp.wait()
            return carry

        lax.fori_loop(0, grp_n, body, 0)
        plsc.subcore_barrier()
        pltpu.sync_copy(shared.at[pl.ds(s * ok, ok)],
                        out_hbm.at[c].at[pl.ds(s * ok, ok)])

    return kern(msg, idx2d, zrows)


def _blockdiag8(w):
    z = jnp.zeros((D, D), w.dtype)
    rows = []
    for j in range(8):
        rows.append(jnp.concatenate([w if i == j else z for i in range(8)], axis=1))
    return jnp.concatenate(rows, axis=0)


def kernel(x, edge_index, edge_attr, params):
    n, _ = x.shape
    np_ = n // 8
    e = edge_attr.shape[0]
    p = params

    step = NC * NS * GRP * CH
    epad = ((e + step - 1) // step) * step
    src = jnp.pad(edge_index[0], (0, epad - e))
    dst_s = jnp.pad(edge_index[1], (0, epad - e), constant_values=n)
    dst_g = jnp.pad(edge_index[1], (0, epad - e))
    idx_src = src.reshape(-1, CH)
    idx_dst_s = dst_s.reshape(-1, CH)
    idx_dst_g = dst_g.reshape(-1, CH)
    ea_pad = jnp.pad(edge_attr, ((0, epad - e), (0, 0)))
    nt_rows = ((n + NS * 8 - 1) // (NS * 8)) * 8
    zrows = jnp.zeros((nt_rows, D), F32)

    w1b = _b16(p['nn_W1'])
    w2b = _b16(p['nn_W2'])
    b2row = p['nn_b2'].reshape(1, D * D)
    arep = jnp.repeat(jnp.eye(D, dtype=BF16), D, axis=1)
    csum = jnp.tile(jnp.eye(D, dtype=BF16), (D, 1))
    b1 = p['nn_b1'].reshape(1, D)
    smat = jnp.tile(jnp.eye(D, dtype=F32), (8, 1))
    stmat = smat.T
    eps = jax.random.normal(jax.random.key(42), (n, D), F32)

    tile = 4096
    hp = x.reshape(np_, 8 * D)
    zp = None
    we = _we_call(ea_pad, w1b, b1, w2b, b2row, tile)
    for i in (1, 2, 3, 4):
        h_nodes = hp.reshape(n, D)
        xj = _sc_gather(h_nodes, idx_src)
        msg = _msg_call(we, xj, arep, csum, tile)
        agg = _sc_scatter_add(msg, idx_dst_s, zrows, n)
        aggp = agg.reshape(NC, np_, 8 * D)
        rootbd = _blockdiag8(_b16(p[f'root{i}']))
        bias = jnp.tile(p[f'bias{i}'], 8).reshape(1, 8 * D)
        g16 = p[f'bn{i}_g'].reshape(1, D)
        bb16 = p[f'bn{i}_b'].reshape(1, D)
        if i < 4:
            hp = _upd_call(aggp, hp, rootbd, bias, g16, bb16, smat, stmat)
        else:
            extra = (_blockdiag8(_b16(p['mu_W'])),
                     jnp.tile(p['mu_b'], 8).reshape(1, 8 * D),
                     _blockdiag8(_b16(p['lv_W'])),
                     jnp.tile(p['lv_b'], 8).reshape(1, 8 * D),
                     eps.reshape(np_, 8 * D))
            zp = _upd_call(aggp, hp, rootbd, bias, g16, bb16, smat, stmat, extra)

    z = zp.reshape(n, D)
    zs = _sc_gather(z, idx_src)
    zd = _sc_gather(z, idx_dst_g)
    ws = [_b16(p['dec_W0'][:D]), _b16(p['dec_W0'][D:]), p['dec_b0'].reshape(1, -1),
          _b16(p['dec_W1']), p['dec_b1'].reshape(1, -1),
          _b16(p['dec_W2']), p['dec_b2'].reshape(1, -1),
          _b16(p['dec_W3']), p['dec_b3'].reshape(1, -1),
          _b16(p['dec_W4']), p['dec_b4'].reshape(1, -1)]
    return _dec_call(zs, zd, ws, tile)[:e]
